# Initial kernel scaffold; baseline (speedup 1.0000x reference)
import functools

import jax
import jax.numpy as jnp
from jax import lax
from jax.experimental import pallas as pl
from jax.experimental.pallas import tpu as pltpu
from jax.experimental.pallas import tpu_sc as plsc

NC = 2
NS = 16
NW = NC * NS
CW = 128


def _ceil_to(v, m):
    return (v + m - 1) // m * m


@functools.lru_cache(maxsize=None)
def _sc_aggregate(np_, ch):
    rpt = np_ // NS
    kz = rpt // CW
    rem = rpt % CW

    def body(h, srcp, dstp, zrow, ones16, z16,
             agg, cnt,
             agg_sh, cnt_sh, src_v, dst_v, rb0, rb1, ones_v, z16_v, zrow_v,
             sem0, sem1):
        c = lax.axis_index("c")
        s = lax.axis_index("s")
        wid = c * NS + s

        pltpu.sync_copy(srcp.at[wid], src_v)
        pltpu.sync_copy(dstp.at[wid], dst_v)
        pltpu.sync_copy(zrow, zrow_v)
        pltpu.sync_copy(ones16, ones_v)
        pltpu.sync_copy(z16, z16_v)

        base = s * rpt
        for k in range(kz):
            pltpu.sync_copy(zrow_v, agg_sh.at[pl.ds(base + k * CW, CW)])
            pltpu.sync_copy(z16_v, cnt_sh.at[pl.ds(base + k * CW, CW)])
        if rem:
            pltpu.sync_copy(zrow_v.at[pl.ds(0, rem)],
                            agg_sh.at[pl.ds(base + kz * CW, rem)])
            pltpu.sync_copy(z16_v.at[pl.ds(0, rem)],
                            cnt_sh.at[pl.ds(base + kz * CW, rem)])
        plsc.subcore_barrier()

        def process(j, rb, sem):
            pltpu.make_async_copy(h.at[src_v.at[j]], rb, sem).wait()
            pltpu.sync_copy(rb, agg_sh.at[dst_v.at[j]], add=True)
            pltpu.sync_copy(ones_v, cnt_sh.at[dst_v.at[j]], add=True)

        pltpu.async_copy(h.at[src_v.at[0]], rb0, sem0)
        pltpu.async_copy(h.at[src_v.at[1]], rb1, sem1)

        def loop_body(i, carry):
            j = 2 * i
            process(j, rb0, sem0)
            pltpu.async_copy(h.at[src_v.at[j + 2]], rb0, sem0)
            process(j + 1, rb1, sem1)
            pltpu.async_copy(h.at[src_v.at[j + 3]], rb1, sem1)
            return carry

        lax.fori_loop(0, ch // 2 - 1, loop_body, 0)
        process(ch - 2, rb0, sem0)
        process(ch - 1, rb1, sem1)
        plsc.subcore_barrier()

        def emit(r0, rows):
            pltpu.sync_copy(agg_sh.at[pl.ds(r0, rows)], rb0.at[pl.ds(0, rows)])
            pltpu.sync_copy(rb0.at[pl.ds(0, rows)], agg.at[c, pl.ds(r0, rows)])
            pltpu.sync_copy(cnt_sh.at[pl.ds(r0, rows)], z16_v.at[pl.ds(0, rows)])
            pltpu.sync_copy(z16_v.at[pl.ds(0, rows)], cnt.at[c, pl.ds(r0, rows)])

        for k in range(kz):
            emit(base + k * CW, CW)
        if rem:
            emit(base + kz * CW, rem)

    return pl.kernel(
        body,
        out_type=(
            jax.ShapeDtypeStruct((NC, np_, 128), jnp.float32),
            jax.ShapeDtypeStruct((NC, np_, 16), jnp.float32),
        ),
        mesh=plsc.VectorSubcoreMesh(core_axis_name="c", subcore_axis_name="s"),
        scratch_types=[
            pltpu.VMEM_SHARED((np_, 128), jnp.float32),
            pltpu.VMEM_SHARED((np_, 16), jnp.float32),
            pltpu.VMEM((ch, CW), jnp.int32),
            pltpu.VMEM((ch, CW), jnp.int32),
            pltpu.VMEM((CW, 128), jnp.float32),
            pltpu.VMEM((CW, 128), jnp.float32),
            pltpu.VMEM((CW, 16), jnp.float32),
            pltpu.VMEM((CW, 16), jnp.float32),
            pltpu.VMEM((CW, 128), jnp.float32),
            pltpu.SemaphoreType.DMA,
            pltpu.SemaphoreType.DMA,
        ],
    )


@functools.lru_cache(maxsize=None)
def _tc_combine(np_, relu):
    blk = 512

    def body(agga, aggb, cnta, cntb, h, wl, wr, b, out):
        cnt = cnta[0, :, 0:1] + cntb[0, :, 0:1]
        inv = 1.0 / jnp.maximum(cnt, 1.0)
        mean = (agga[0] + aggb[0]) * inv
        acc = lax.dot_general(mean, wl[...], (((1,), (1,)), ((), ())),
                              preferred_element_type=jnp.float32)
        acc = acc + lax.dot_general(h[...], wr[...], (((1,), (1,)), ((), ())),
                                    preferred_element_type=jnp.float32)
        acc = acc + b[...]
        out[...] = jnp.maximum(acc, 0.0) if relu else acc

    return pl.pallas_call(
        body,
        grid=(np_ // blk,),
        in_specs=[
            pl.BlockSpec((1, blk, 128), lambda i: (0, i, 0)),
            pl.BlockSpec((1, blk, 128), lambda i: (1, i, 0)),
            pl.BlockSpec((1, blk, 16), lambda i: (0, i, 0)),
            pl.BlockSpec((1, blk, 16), lambda i: (1, i, 0)),
            pl.BlockSpec((blk, 128), lambda i: (i, 0)),
            pl.BlockSpec((128, 128), lambda i: (0, 0)),
            pl.BlockSpec((128, 128), lambda i: (0, 0)),
            pl.BlockSpec((1, 128), lambda i: (0, 0)),
        ],
        out_specs=pl.BlockSpec((blk, 128), lambda i: (i, 0)),
        out_shape=jax.ShapeDtypeStruct((np_, 128), jnp.float32),
    )


def kernel(x, edge_index, Wl1, bl1, Wr1, Wl2, bl2, Wr2):
    n, d = x.shape
    e = edge_index.shape[1]

    np_ = _ceil_to(n + 1, 512)
    ept = _ceil_to(-(-e // NW), 2 * CW)
    ch = ept // CW

    src = edge_index[0]
    dst = edge_index[1]
    pad_e = NW * ept - e
    srcp = jnp.concatenate([src, jnp.zeros((pad_e,), jnp.int32)]).reshape(NW, ch, CW)
    dstp = jnp.concatenate([dst, jnp.full((pad_e,), n, jnp.int32)]).reshape(NW, ch, CW)
    xp = jnp.pad(x, ((0, np_ - n), (0, 0)))

    zrow = jnp.zeros((CW, 128), jnp.float32)
    ones16 = jnp.ones((CW, 16), jnp.float32)
    z16 = jnp.zeros((CW, 16), jnp.float32)

    sc = _sc_aggregate(np_, ch)
    b1 = bl1.reshape(1, 128)
    b2 = bl2.reshape(1, 128)

    agg1, cnt1 = sc(xp, srcp, dstp, zrow, ones16, z16)
    h1 = _tc_combine(np_, True)(agg1, agg1, cnt1, cnt1, xp, Wl1, Wr1, b1)
    agg2, cnt2 = sc(h1, srcp, dstp, zrow, ones16, z16)
    h2 = _tc_combine(np_, False)(agg2, agg2, cnt2, cnt2, h1, Wl2, Wr2, b2)
    return h2[:n]


# speedup vs baseline: 6.7252x; 6.7252x over previous
import functools

import jax
import jax.numpy as jnp
from jax import lax
from jax.experimental import pallas as pl
from jax.experimental.pallas import tpu as pltpu
from jax.experimental.pallas import tpu_sc as plsc

NC = 2
NS = 16
CW = 128
HD = 64


def _ceil_to(v, m):
    return (v + m - 1) // m * m


@functools.lru_cache(maxsize=None)
def _sc_aggregate(np_, ch):
    rpt = np_ // NS
    kz = rpt // CW
    rem = rpt % CW

    def body(h, srcp, dstp, zrow, ones16, z16,
             agg, cnt,
             agg_sh, cnt_sh, src_v, dst_v, rb0, rb1, ones_v, z16_v, zrow_v,
             sem0, sem1):
        c = lax.axis_index("c")
        s = lax.axis_index("s")

        pltpu.sync_copy(srcp.at[c, s], src_v)
        pltpu.sync_copy(dstp.at[s], dst_v)
        pltpu.sync_copy(zrow, zrow_v)
        pltpu.sync_copy(ones16, ones_v)
        pltpu.sync_copy(z16, z16_v)

        base = s * rpt
        for k in range(kz):
            pltpu.sync_copy(zrow_v, agg_sh.at[pl.ds(base + k * CW, CW)])
            pltpu.sync_copy(z16_v, cnt_sh.at[pl.ds(base + k * CW, CW)])
        if rem:
            pltpu.sync_copy(zrow_v.at[pl.ds(0, rem)],
                            agg_sh.at[pl.ds(base + kz * CW, rem)])
            pltpu.sync_copy(z16_v.at[pl.ds(0, rem)],
                            cnt_sh.at[pl.ds(base + kz * CW, rem)])
        plsc.subcore_barrier()

        def process(j, rb, sem):
            pltpu.make_async_copy(h.at[src_v.at[j]], rb, sem).wait()
            pltpu.sync_copy(rb, agg_sh.at[dst_v.at[j]], add=True)
            pltpu.sync_copy(ones_v, cnt_sh.at[dst_v.at[j]], add=True)

        pltpu.async_copy(h.at[src_v.at[0]], rb0, sem0)
        pltpu.async_copy(h.at[src_v.at[1]], rb1, sem1)

        def loop_body(i, carry):
            j = 2 * i
            process(j, rb0, sem0)
            pltpu.async_copy(h.at[src_v.at[j + 2]], rb0, sem0)
            process(j + 1, rb1, sem1)
            pltpu.async_copy(h.at[src_v.at[j + 3]], rb1, sem1)
            return carry

        lax.fori_loop(0, ch // 2 - 1, loop_body, 0)
        process(ch - 2, rb0, sem0)
        process(ch - 1, rb1, sem1)
        plsc.subcore_barrier()

        def emit_agg(r0, rows):
            pltpu.sync_copy(agg_sh.at[pl.ds(r0, rows)], rb0.at[pl.ds(0, rows)])
            pltpu.sync_copy(rb0.at[pl.ds(0, rows)], agg.at[c, pl.ds(r0, rows)])

        def emit_cnt(r0, rows):
            pltpu.sync_copy(cnt_sh.at[pl.ds(r0, rows)], z16_v.at[pl.ds(0, rows)])
            pltpu.sync_copy(z16_v.at[pl.ds(0, rows)], cnt.at[pl.ds(r0, rows)])

        for k in range(kz):
            emit_agg(base + k * CW, CW)
        if rem:
            emit_agg(base + kz * CW, rem)

        @pl.when(c == 0)
        def _():
            for k in range(kz):
                emit_cnt(base + k * CW, CW)
            if rem:
                emit_cnt(base + kz * CW, rem)

    return pl.kernel(
        body,
        out_type=(
            jax.ShapeDtypeStruct((NC, np_, HD), jnp.float32),
            jax.ShapeDtypeStruct((np_, 16), jnp.float32),
        ),
        mesh=plsc.VectorSubcoreMesh(core_axis_name="c", subcore_axis_name="s",
                                    num_cores=NC, num_subcores=NS),
        compiler_params=pltpu.CompilerParams(use_tc_tiling_on_sc=False),
        scratch_types=[
            pltpu.VMEM_SHARED((np_, HD), jnp.float32),
            pltpu.VMEM_SHARED((np_, 16), jnp.float32),
            pltpu.VMEM((ch, CW), jnp.int32),
            pltpu.VMEM((ch, CW), jnp.int32),
            pltpu.VMEM((CW, HD), jnp.float32),
            pltpu.VMEM((CW, HD), jnp.float32),
            pltpu.VMEM((CW, 16), jnp.float32),
            pltpu.VMEM((CW, 16), jnp.float32),
            pltpu.VMEM((CW, HD), jnp.float32),
            pltpu.SemaphoreType.DMA,
            pltpu.SemaphoreType.DMA,
        ],
    )


@functools.lru_cache(maxsize=None)
def _tc_combine(np_, relu, split_out):
    blk = 512

    def body(agg, cnt, h, wl, wr, b, out):
        inv = 1.0 / jnp.maximum(cnt[:, 0:1], 1.0)
        mean = jnp.concatenate([agg[0], agg[1]], axis=1) * inv
        hb = jnp.concatenate([h[0], h[1]], axis=1)
        acc = lax.dot_general(mean, wl[...], (((1,), (1,)), ((), ())),
                              preferred_element_type=jnp.float32)
        acc = acc + lax.dot_general(hb, wr[...], (((1,), (1,)), ((), ())),
                                    preferred_element_type=jnp.float32)
        acc = acc + b[...]
        if relu:
            acc = jnp.maximum(acc, 0.0)
        if split_out:
            out[0] = acc[:, :HD]
            out[1] = acc[:, HD:]
        else:
            out[...] = acc

    if split_out:
        out_spec = pl.BlockSpec((NC, blk, HD), lambda i: (0, i, 0))
        out_shape = jax.ShapeDtypeStruct((NC, np_, HD), jnp.float32)
    else:
        out_spec = pl.BlockSpec((blk, 128), lambda i: (i, 0))
        out_shape = jax.ShapeDtypeStruct((np_, 128), jnp.float32)

    return pl.pallas_call(
        body,
        grid=(np_ // blk,),
        in_specs=[
            pl.BlockSpec((NC, blk, HD), lambda i: (0, i, 0)),
            pl.BlockSpec((blk, 16), lambda i: (i, 0)),
            pl.BlockSpec((NC, blk, HD), lambda i: (0, i, 0)),
            pl.BlockSpec((128, 128), lambda i: (0, 0)),
            pl.BlockSpec((128, 128), lambda i: (0, 0)),
            pl.BlockSpec((1, 128), lambda i: (0, 0)),
        ],
        out_specs=out_spec,
        out_shape=out_shape,
    )


def kernel(x, edge_index, Wl1, bl1, Wr1, Wl2, bl2, Wr2):
    n, d = x.shape
    e = edge_index.shape[1]

    np_ = _ceil_to(n + 1, 512)
    ept = _ceil_to(-(-e // NS), 2 * CW)
    ch = ept // CW

    src = edge_index[0]
    dst = edge_index[1]
    pad_e = NS * ept - e
    src_t = jnp.concatenate([src, jnp.zeros((pad_e,), jnp.int32)]).reshape(NS, ch, CW)
    srcp = jnp.stack([src_t, src_t + np_])
    dstp = jnp.concatenate([dst, jnp.full((pad_e,), n, jnp.int32)]).reshape(NS, ch, CW)
    xs = jnp.pad(x, ((0, np_ - n), (0, 0))).reshape(np_, NC, HD).transpose(1, 0, 2)

    zrow = jnp.zeros((CW, HD), jnp.float32)
    ones16 = jnp.ones((CW, 16), jnp.float32)
    z16 = jnp.zeros((CW, 16), jnp.float32)

    sc = _sc_aggregate(np_, ch)
    b1 = bl1.reshape(1, 128)
    b2 = bl2.reshape(1, 128)

    def flat(hs):
        return hs.reshape(NC * np_, HD)

    agg1, cnt1 = sc(flat(xs), srcp, dstp, zrow, ones16, z16)
    h1 = _tc_combine(np_, True, True)(agg1, cnt1, xs, Wl1, Wr1, b1)
    agg2, cnt2 = sc(flat(h1), srcp, dstp, zrow, ones16, z16)
    h2 = _tc_combine(np_, False, False)(agg2, cnt2, h1, Wl2, Wr2, b2)
    return h2[:n]
